# baseline (speedup 1.0000x reference)
import functools

import jax
import jax.numpy as jnp
from jax import lax
from jax.experimental import pallas as pl
from jax.experimental.pallas import tpu as pltpu
from jax.experimental.pallas import tpu_sc as plsc

_NC = 2
_NS = 16
_NW = _NC * _NS


def _sc_body(w_hbm, out_hbm, sem, *, bsz, rows):
    wid = lax.axis_index("s") * _NC + lax.axis_index("c")
    base = wid * rows
    copies = [
        pltpu.async_copy(
            w_hbm.at[pl.ds(base, rows)],
            out_hbm.at[b].at[pl.ds(base, rows)],
            sem,
        )
        for b in range(bsz)
    ]
    for c in copies:
        c.wait()


def kernel(input_tensor, weight):
    bsz, seq_len, dim = input_tensor.shape
    rows = seq_len // _NW
    mesh = plsc.VectorSubcoreMesh(core_axis_name="c", subcore_axis_name="s")
    body = functools.partial(_sc_body, bsz=bsz, rows=rows)
    return pl.kernel(
        body,
        mesh=mesh,
        out_type=jax.ShapeDtypeStruct((bsz, seq_len, dim), weight.dtype),
        scratch_types=[pltpu.SemaphoreType.DMA],
    )(weight[:seq_len])


# speedup vs baseline: 52.3265x; 52.3265x over previous
import functools

import jax
import jax.numpy as jnp
from jax import lax
from jax.experimental import pallas as pl
from jax.experimental.pallas import tpu as pltpu
from jax.experimental.pallas import tpu_sc as plsc

_NC = 2
_NS = 16
_NW = _NC * _NS
_CHUNK = 32


def _sc_body(w_hbm, out_hbm, buf0, buf1, rsem, wsem0, wsem1, *, bsz, rows):
    wid = lax.axis_index("s") * _NC + lax.axis_index("c")
    base = wid * rows
    nchunks = rows // _CHUNK
    bufs = (buf0, buf1)
    wsems = (wsem0, wsem1)
    writes = [None] * nchunks
    for g in range(nchunks):
        buf = bufs[g % 2]
        if g >= 2:
            for c in writes[g - 2]:
                c.wait()
        start = base + g * _CHUNK
        pltpu.async_copy(w_hbm.at[pl.ds(start, _CHUNK)], buf, rsem).wait()
        writes[g] = [
            pltpu.async_copy(buf, out_hbm.at[b].at[pl.ds(start, _CHUNK)], wsems[g % 2])
            for b in range(bsz)
        ]
    for g in range(max(nchunks - 2, 0), nchunks):
        for c in writes[g]:
            c.wait()


def kernel(input_tensor, weight):
    bsz, seq_len, dim = input_tensor.shape
    rows = seq_len // _NW
    mesh = plsc.VectorSubcoreMesh(core_axis_name="c", subcore_axis_name="s")
    body = functools.partial(_sc_body, bsz=bsz, rows=rows)
    return pl.kernel(
        body,
        mesh=mesh,
        out_type=jax.ShapeDtypeStruct((bsz, seq_len, dim), weight.dtype),
        scratch_types=[
            pltpu.VMEM((_CHUNK, dim), jnp.float32),
            pltpu.VMEM((_CHUNK, dim), jnp.float32),
            pltpu.SemaphoreType.DMA,
            pltpu.SemaphoreType.DMA,
            pltpu.SemaphoreType.DMA,
        ],
    )(weight[:seq_len])
